# baseline (speedup 1.0000x reference)
import functools

import jax
import jax.numpy as jnp
from jax import lax
from jax.experimental import pallas as pl
from jax.experimental.pallas import tpu as pltpu
from jax.experimental.pallas import tpu_sc as plsc

_N = 64
_R = 576
_D = 64
_K = 1024
_NTOK = _N * _R

_BM = 1024
_GRID = _NTOK // _BM

_NB = 4
_LGRID = _N // _NB

_LAM_VQ = 1.0
_LAM_COMMIT = 0.5
_LAM_ORTH = 0.0

_EPS32 = 1.1920929e-07
_TOLC = (_K * _EPS32) ** 2


def _argmin_body(x_ref, w_ref, o_ref):
    x = x_ref[...]
    w = w_ref[...]
    m2 = lax.dot_general(x, w * -2.0, (((1,), (0,)), ((), ())),
                         preferred_element_type=jnp.float32)
    x2 = jnp.sum(x * x, axis=1, keepdims=True)
    e2 = jnp.sum(w * w, axis=0, keepdims=True)
    dist = x2 + m2 + e2
    mn = jnp.min(dist, axis=1, keepdims=True)
    lanef = lax.broadcasted_iota(jnp.int32, dist.shape, 1).astype(jnp.float32)
    idxf = jnp.min(jnp.where(dist == mn, lanef, jnp.float32(_K)), axis=1)
    o_ref[...] = idxf.astype(jnp.int32).reshape(_BM // 128, 128)


def _argmin_call(flat, weight):
    ntok = flat.shape[0]
    idx2 = pl.pallas_call(
        _argmin_body,
        grid=(ntok // _BM,),
        in_specs=[
            pl.BlockSpec((_BM, _D), lambda i: (i, 0)),
            pl.BlockSpec((_D, _K), lambda i: (0, 0)),
        ],
        out_specs=pl.BlockSpec((_BM // 128, 128), lambda i: (i, 0)),
        out_shape=jax.ShapeDtypeStruct((ntok // 128, 128), jnp.int32),
    )(flat, weight)
    return idx2.reshape(-1)


def _sc_gather(emb, idx):
    info = plsc.get_sparse_core_info()
    nc, ns = info.num_cores, info.num_subcores
    nw = nc * ns
    b = idx.shape[0]
    b_per_w = b // nw
    ch = 128
    n_ch = -(-b_per_w // ch)
    mesh = plsc.VectorSubcoreMesh(core_axis_name="c", subcore_axis_name="s")

    nb_w = b_per_w // _R

    @functools.partial(
        pl.kernel, mesh=mesh,
        compiler_params=pltpu.CompilerParams(use_tc_tiling_on_sc=False),
        out_type=[jax.ShapeDtypeStruct((b, _D), jnp.float32),
                  jax.ShapeDtypeStruct((b // _R, _R), jnp.int32)],
        scratch_types=[
            pltpu.VMEM((b_per_w,), jnp.int32),
            pltpu.VMEM((nb_w, _R), jnp.int32),
            pltpu.VMEM((b_per_w, _D), jnp.float32),
            pltpu.SemaphoreType.DMA,
        ],
    )
    def k(emb_hbm, idx_hbm, out_hbm, fidx_hbm, idx_v, idx_v2, rows_v, sem):
        wid = lax.axis_index("s") * nc + lax.axis_index("c")
        base = wid * b_per_w
        pltpu.sync_copy(idx_hbm.at[pl.ds(base, b_per_w)], idx_v)
        copies = []
        for j in range(n_ch):
            lo = j * ch
            sz = min(ch, b_per_w - lo)
            copies.append(pltpu.async_copy(
                emb_hbm.at[idx_v.at[pl.ds(lo, sz)]],
                rows_v.at[pl.ds(lo, sz)], sem))
        for r in range(nb_w):
            pltpu.sync_copy(idx_hbm.at[pl.ds(base + r * _R, _R)], idx_v2.at[r])
        pltpu.sync_copy(idx_v2, fidx_hbm.at[pl.ds(wid * nb_w, nb_w)])
        for c in copies:
            c.wait()
        pltpu.sync_copy(rows_v, out_hbm.at[pl.ds(base, b_per_w)])

    return k(emb, idx)


def _rankorth_body(w_ref, orth_ref, rank_ref):
    w = w_ref[...]
    g = lax.dot_general(w, w, (((1,), (1,)), ((), ())),
                        precision=lax.Precision.HIGHEST,
                        preferred_element_type=jnp.float32)
    subl = lax.broadcasted_iota(jnp.int32, (_D, _D), 0)
    lane = lax.broadcasted_iota(jnp.int32, (_D, _D), 1)
    eye = jnp.where(subl == lane, 1.0, 0.0).astype(jnp.float32)
    gi = g - eye
    orth_ref[0, 0] = jnp.sqrt(jnp.sum(gi * gi))

    def piter(_, carry):
        vrow, lam = carry
        vcol = jnp.sum(g * vrow, axis=1, keepdims=True)
        vcol = vcol / jnp.max(jnp.abs(vcol))
        vrow2 = jnp.sum(g * vcol, axis=0, keepdims=True)
        lam2 = jnp.max(jnp.abs(vrow2))
        return (vrow2 / lam2, lam2)
    _, lam = lax.fori_loop(
        0, 8, piter, (jnp.ones((1, _D), jnp.float32),
                      jnp.asarray(1.0, jnp.float32)))

    tau = lam * _TOLC
    a0 = g - tau * eye
    lane1 = lax.broadcasted_iota(jnp.int32, (1, _D), 1)
    sub1 = lax.broadcasted_iota(jnp.int32, (_D, 1), 0)

    def ldl_step(j, carry):
        a, cnt = carry
        lanej = lane == j
        sublj = subl == j
        d = jnp.sum(jnp.where(lanej & sublj, a, 0.0))
        row = jnp.sum(jnp.where(sublj, a, 0.0), axis=0, keepdims=True)
        col = jnp.sum(jnp.where(lanej, a, 0.0), axis=1, keepdims=True)
        rowm = jnp.where(lane1 > j, row, 0.0)
        colm = jnp.where(sub1 > j, col, 0.0)
        dsafe = jnp.where(d == 0.0, jnp.asarray(-1e-30, jnp.float32), d)
        a = a - colm * (rowm / dsafe)
        cnt = cnt + (d > 0.0).astype(jnp.int32)
        return (a, cnt)
    _, cnt = lax.fori_loop(0, _D, ldl_step, (a0, jnp.asarray(0, jnp.int32)))
    rank_ref[0, 0] = cnt.astype(jnp.float32)


def _rankorth_call(weight):
    scal = jax.ShapeDtypeStruct((1, 1), jnp.float32)
    smem_out = pl.BlockSpec(memory_space=pltpu.SMEM)
    return pl.pallas_call(
        _rankorth_body,
        out_specs=[smem_out, smem_out],
        out_shape=[scal, scal],
    )(weight)


def _loss_step(x_ref, q_ref, d4_sqrt_acc):
    diff = q_ref[...] - x_ref[...]
    d2 = diff * diff
    d4 = d2 * d2
    c = jnp.float32(0.0)
    for bi in range(_NB):
        s = jnp.sum(d4[bi * _R:(bi + 1) * _R], axis=0)
        c = c + jnp.sum(jnp.sqrt(s))
    d4_sqrt_acc[0] += c


def _loss_part_body(x_ref, q_ref, accin_ref, accout_ref, acc_ref):
    step = pl.program_id(0)

    @pl.when(step == 0)
    def _init():
        acc_ref[0] = accin_ref[0, 0]

    _loss_step(x_ref, q_ref, acc_ref)

    @pl.when(step == pl.num_programs(0) - 1)
    def _fin():
        accout_ref[0, 0] = acc_ref[0]


def _loss_final_body(x_ref, q_ref, accin_ref, orth_ref, vq_ref, ql_ref, acc_ref):
    step = pl.program_id(0)

    @pl.when(step == 0)
    def _init():
        acc_ref[0] = accin_ref[0, 0]

    _loss_step(x_ref, q_ref, acc_ref)

    @pl.when(step == pl.num_programs(0) - 1)
    def _fin():
        vq = acc_ref[0] / (_N * _D)
        vq_ref[0, 0] = vq
        ql_ref[0, 0] = _LAM_VQ * vq + _LAM_COMMIT * vq + _LAM_ORTH * orth_ref[0, 0]


def _loss_call(flat, quant_part, row0, acc_in, orth=None):
    rows = quant_part.shape[0]
    grid = rows // (_NB * _R)
    blk0 = row0 // (_NB * _R)
    scal = jax.ShapeDtypeStruct((1, 1), jnp.float32)
    smem = pl.BlockSpec(memory_space=pltpu.SMEM)
    in_specs = [
        pl.BlockSpec((_NB * _R, _D), lambda i: (i + blk0, 0)),
        pl.BlockSpec((_NB * _R, _D), lambda i: (i, 0)),
        smem,
    ]
    if orth is None:
        return pl.pallas_call(
            _loss_part_body,
            grid=(grid,),
            in_specs=in_specs,
            out_specs=smem,
            out_shape=scal,
            scratch_shapes=[pltpu.SMEM((1,), jnp.float32)],
        )(flat, quant_part, acc_in)
    return pl.pallas_call(
        _loss_final_body,
        grid=(grid,),
        in_specs=in_specs + [smem],
        out_specs=[smem, smem],
        out_shape=[scal, scal],
        scratch_shapes=[pltpu.SMEM((1,), jnp.float32)],
    )(flat, quant_part, acc_in, orth)


def kernel(soft_fillers, weight):
    n, r, d = soft_fillers.shape
    flat = soft_fillers.reshape(n * r, d)
    emb = weight.T
    idx = _argmin_call(flat, weight)
    quant, fidx = _sc_gather(emb, idx)
    orth, rank = _rankorth_call(weight)
    zero = jnp.zeros((1, 1), jnp.float32)
    vq, ql = _loss_call(flat, quant, 0, zero, orth)
    quant3 = quant.reshape(n, r, d)
    return (ql[0, 0], vq[0, 0], vq[0, 0], orth[0, 0], rank[0, 0],
            quant3, quant3, fidx)


# speedup vs baseline: 1.0282x; 1.0282x over previous
import functools

import jax
import jax.numpy as jnp
from jax import lax
from jax.experimental import pallas as pl
from jax.experimental.pallas import tpu as pltpu
from jax.experimental.pallas import tpu_sc as plsc

_N = 64
_R = 576
_D = 64
_K = 1024
_NTOK = _N * _R

_BM = 2048
_GRID = _NTOK // _BM

_NB = 4
_LGRID = _N // _NB

_LAM_VQ = 1.0
_LAM_COMMIT = 0.5
_LAM_ORTH = 0.0

_EPS32 = 1.1920929e-07
_TOLC = (_K * _EPS32) ** 2


def _argmin_body(x_ref, w_ref, o_ref):
    x = x_ref[...]
    w = w_ref[...]
    m2 = lax.dot_general(x, w * -2.0, (((1,), (0,)), ((), ())),
                         preferred_element_type=jnp.float32)
    x2 = jnp.sum(x * x, axis=1, keepdims=True)
    e2 = jnp.sum(w * w, axis=0, keepdims=True)
    dist = x2 + m2 + e2
    mn = jnp.min(dist, axis=1, keepdims=True)
    lanef = lax.broadcasted_iota(jnp.int32, dist.shape, 1).astype(jnp.float32)
    idxf = jnp.min(jnp.where(dist == mn, lanef, jnp.float32(_K)), axis=1)
    o_ref[...] = idxf.astype(jnp.int32).reshape(_BM // 128, 128)


def _argmin_call(flat, weight):
    ntok = flat.shape[0]
    idx2 = pl.pallas_call(
        _argmin_body,
        grid=(ntok // _BM,),
        in_specs=[
            pl.BlockSpec((_BM, _D), lambda i: (i, 0)),
            pl.BlockSpec((_D, _K), lambda i: (0, 0)),
        ],
        out_specs=pl.BlockSpec((_BM // 128, 128), lambda i: (i, 0)),
        out_shape=jax.ShapeDtypeStruct((ntok // 128, 128), jnp.int32),
    )(flat, weight)
    return idx2.reshape(-1)


def _sc_gather(emb, idx):
    info = plsc.get_sparse_core_info()
    nc, ns = info.num_cores, info.num_subcores
    nw = nc * ns
    b = idx.shape[0]
    b_per_w = b // nw
    ch = 128
    n_ch = -(-b_per_w // ch)
    mesh = plsc.VectorSubcoreMesh(core_axis_name="c", subcore_axis_name="s")

    nb_w = b_per_w // _R

    @functools.partial(
        pl.kernel, mesh=mesh,
        compiler_params=pltpu.CompilerParams(use_tc_tiling_on_sc=False),
        out_type=[jax.ShapeDtypeStruct((b, _D), jnp.float32),
                  jax.ShapeDtypeStruct((b // _R, _R), jnp.int32)],
        scratch_types=[
            pltpu.VMEM((b_per_w,), jnp.int32),
            pltpu.VMEM((nb_w, _R), jnp.int32),
            pltpu.VMEM((b_per_w, _D), jnp.float32),
            pltpu.SemaphoreType.DMA,
        ],
    )
    def k(emb_hbm, idx_hbm, out_hbm, fidx_hbm, idx_v, idx_v2, rows_v, sem):
        wid = lax.axis_index("s") * nc + lax.axis_index("c")
        base = wid * b_per_w
        pltpu.sync_copy(idx_hbm.at[pl.ds(base, b_per_w)], idx_v)
        copies = []
        for j in range(n_ch):
            lo = j * ch
            sz = min(ch, b_per_w - lo)
            copies.append(pltpu.async_copy(
                emb_hbm.at[idx_v.at[pl.ds(lo, sz)]],
                rows_v.at[pl.ds(lo, sz)], sem))
        for r in range(nb_w):
            pltpu.sync_copy(idx_hbm.at[pl.ds(base + r * _R, _R)], idx_v2.at[r])
        pltpu.sync_copy(idx_v2, fidx_hbm.at[pl.ds(wid * nb_w, nb_w)])
        for c in copies:
            c.wait()
        pltpu.sync_copy(rows_v, out_hbm.at[pl.ds(base, b_per_w)])

    return k(emb, idx)


def _rankorth_body(w_ref, orth_ref, rank_ref):
    w = w_ref[...]
    g = lax.dot_general(w, w, (((1,), (1,)), ((), ())),
                        precision=lax.Precision.HIGHEST,
                        preferred_element_type=jnp.float32)
    subl = lax.broadcasted_iota(jnp.int32, (_D, _D), 0)
    lane = lax.broadcasted_iota(jnp.int32, (_D, _D), 1)
    eye = jnp.where(subl == lane, 1.0, 0.0).astype(jnp.float32)
    gi = g - eye
    orth_ref[0, 0] = jnp.sqrt(jnp.sum(gi * gi))

    def piter(_, carry):
        vrow, lam = carry
        vcol = jnp.sum(g * vrow, axis=1, keepdims=True)
        vcol = vcol / jnp.max(jnp.abs(vcol))
        vrow2 = jnp.sum(g * vcol, axis=0, keepdims=True)
        lam2 = jnp.max(jnp.abs(vrow2))
        return (vrow2 / lam2, lam2)
    _, lam = lax.fori_loop(
        0, 8, piter, (jnp.ones((1, _D), jnp.float32),
                      jnp.asarray(1.0, jnp.float32)))

    tau = lam * _TOLC
    a0 = g - tau * eye
    lane1 = lax.broadcasted_iota(jnp.int32, (1, _D), 1)
    sub1 = lax.broadcasted_iota(jnp.int32, (_D, 1), 0)

    def ldl_step(j, carry):
        a, cnt = carry
        lanej = lane == j
        sublj = subl == j
        d = jnp.sum(jnp.where(lanej & sublj, a, 0.0))
        row = jnp.sum(jnp.where(sublj, a, 0.0), axis=0, keepdims=True)
        col = jnp.sum(jnp.where(lanej, a, 0.0), axis=1, keepdims=True)
        rowm = jnp.where(lane1 > j, row, 0.0)
        colm = jnp.where(sub1 > j, col, 0.0)
        dsafe = jnp.where(d == 0.0, jnp.asarray(-1e-30, jnp.float32), d)
        a = a - colm * (rowm / dsafe)
        cnt = cnt + (d > 0.0).astype(jnp.int32)
        return (a, cnt)
    _, cnt = lax.fori_loop(0, _D, ldl_step, (a0, jnp.asarray(0, jnp.int32)))
    rank_ref[0, 0] = cnt.astype(jnp.float32)


def _rankorth_call(weight):
    scal = jax.ShapeDtypeStruct((1, 1), jnp.float32)
    smem_out = pl.BlockSpec(memory_space=pltpu.SMEM)
    return pl.pallas_call(
        _rankorth_body,
        out_specs=[smem_out, smem_out],
        out_shape=[scal, scal],
    )(weight)


def _loss_step(x_ref, q_ref, d4_sqrt_acc):
    diff = q_ref[...] - x_ref[...]
    d2 = diff * diff
    d4 = d2 * d2
    c = jnp.float32(0.0)
    for bi in range(_NB):
        s = jnp.sum(d4[bi * _R:(bi + 1) * _R], axis=0)
        c = c + jnp.sum(jnp.sqrt(s))
    d4_sqrt_acc[0] += c


def _loss_part_body(x_ref, q_ref, accin_ref, accout_ref, acc_ref):
    step = pl.program_id(0)

    @pl.when(step == 0)
    def _init():
        acc_ref[0] = accin_ref[0, 0]

    _loss_step(x_ref, q_ref, acc_ref)

    @pl.when(step == pl.num_programs(0) - 1)
    def _fin():
        accout_ref[0, 0] = acc_ref[0]


def _loss_final_body(x_ref, q_ref, accin_ref, orth_ref, vq_ref, ql_ref, acc_ref):
    step = pl.program_id(0)

    @pl.when(step == 0)
    def _init():
        acc_ref[0] = accin_ref[0, 0]

    _loss_step(x_ref, q_ref, acc_ref)

    @pl.when(step == pl.num_programs(0) - 1)
    def _fin():
        vq = acc_ref[0] / (_N * _D)
        vq_ref[0, 0] = vq
        ql_ref[0, 0] = _LAM_VQ * vq + _LAM_COMMIT * vq + _LAM_ORTH * orth_ref[0, 0]


def _loss_call(flat, quant_part, row0, acc_in, orth=None):
    rows = quant_part.shape[0]
    grid = rows // (_NB * _R)
    blk0 = row0 // (_NB * _R)
    scal = jax.ShapeDtypeStruct((1, 1), jnp.float32)
    smem = pl.BlockSpec(memory_space=pltpu.SMEM)
    in_specs = [
        pl.BlockSpec((_NB * _R, _D), lambda i: (i + blk0, 0)),
        pl.BlockSpec((_NB * _R, _D), lambda i: (i, 0)),
        smem,
    ]
    if orth is None:
        return pl.pallas_call(
            _loss_part_body,
            grid=(grid,),
            in_specs=in_specs,
            out_specs=smem,
            out_shape=scal,
            scratch_shapes=[pltpu.SMEM((1,), jnp.float32)],
        )(flat, quant_part, acc_in)
    return pl.pallas_call(
        _loss_final_body,
        grid=(grid,),
        in_specs=in_specs + [smem],
        out_specs=[smem, smem],
        out_shape=[scal, scal],
        scratch_shapes=[pltpu.SMEM((1,), jnp.float32)],
    )(flat, quant_part, acc_in, orth)


def kernel(soft_fillers, weight):
    n, r, d = soft_fillers.shape
    flat = soft_fillers.reshape(n * r, d)
    emb = weight.T
    idx = _argmin_call(flat, weight)
    quant, fidx = _sc_gather(emb, idx)
    orth, rank = _rankorth_call(weight)
    zero = jnp.zeros((1, 1), jnp.float32)
    vq, ql = _loss_call(flat, quant, 0, zero, orth)
    quant3 = quant.reshape(n, r, d)
    return (ql[0, 0], vq[0, 0], vq[0, 0], orth[0, 0], rank[0, 0],
            quant3, quant3, fidx)


# speedup vs baseline: 1.0378x; 1.0093x over previous
import functools

import jax
import jax.numpy as jnp
from jax import lax
from jax.experimental import pallas as pl
from jax.experimental.pallas import tpu as pltpu
from jax.experimental.pallas import tpu_sc as plsc

_N = 64
_R = 576
_D = 64
_K = 1024
_NTOK = _N * _R

_BM = 4096
_GRID = _NTOK // _BM

_NB = 4
_LGRID = _N // _NB

_LAM_VQ = 1.0
_LAM_COMMIT = 0.5
_LAM_ORTH = 0.0

_EPS32 = 1.1920929e-07
_TOLC = (_K * _EPS32) ** 2


def _argmin_body(x_ref, w_ref, o_ref):
    x = x_ref[...]
    w = w_ref[...]
    m2 = lax.dot_general(x, w * -2.0, (((1,), (0,)), ((), ())),
                         preferred_element_type=jnp.float32)
    x2 = jnp.sum(x * x, axis=1, keepdims=True)
    e2 = jnp.sum(w * w, axis=0, keepdims=True)
    dist = x2 + m2 + e2
    mn = jnp.min(dist, axis=1, keepdims=True)
    lanef = lax.broadcasted_iota(jnp.int32, dist.shape, 1).astype(jnp.float32)
    idxf = jnp.min(jnp.where(dist == mn, lanef, jnp.float32(_K)), axis=1)
    o_ref[...] = idxf.astype(jnp.int32).reshape(_BM // 128, 128)


def _argmin_call(flat, weight):
    ntok = flat.shape[0]
    idx2 = pl.pallas_call(
        _argmin_body,
        grid=(ntok // _BM,),
        in_specs=[
            pl.BlockSpec((_BM, _D), lambda i: (i, 0)),
            pl.BlockSpec((_D, _K), lambda i: (0, 0)),
        ],
        out_specs=pl.BlockSpec((_BM // 128, 128), lambda i: (i, 0)),
        out_shape=jax.ShapeDtypeStruct((ntok // 128, 128), jnp.int32),
    )(flat, weight)
    return idx2.reshape(-1)


def _sc_gather(emb, idx):
    info = plsc.get_sparse_core_info()
    nc, ns = info.num_cores, info.num_subcores
    nw = nc * ns
    b = idx.shape[0]
    b_per_w = b // nw
    ch = 128
    n_ch = -(-b_per_w // ch)
    mesh = plsc.VectorSubcoreMesh(core_axis_name="c", subcore_axis_name="s")

    nb_w = b_per_w // _R

    @functools.partial(
        pl.kernel, mesh=mesh,
        compiler_params=pltpu.CompilerParams(use_tc_tiling_on_sc=False),
        out_type=[jax.ShapeDtypeStruct((b, _D), jnp.float32),
                  jax.ShapeDtypeStruct((b // _R, _R), jnp.int32)],
        scratch_types=[
            pltpu.VMEM((b_per_w,), jnp.int32),
            pltpu.VMEM((nb_w, _R), jnp.int32),
            pltpu.VMEM((b_per_w, _D), jnp.float32),
            pltpu.SemaphoreType.DMA,
        ],
    )
    def k(emb_hbm, idx_hbm, out_hbm, fidx_hbm, idx_v, idx_v2, rows_v, sem):
        wid = lax.axis_index("s") * nc + lax.axis_index("c")
        base = wid * b_per_w
        pltpu.sync_copy(idx_hbm.at[pl.ds(base, b_per_w)], idx_v)
        copies = []
        for j in range(n_ch):
            lo = j * ch
            sz = min(ch, b_per_w - lo)
            copies.append(pltpu.async_copy(
                emb_hbm.at[idx_v.at[pl.ds(lo, sz)]],
                rows_v.at[pl.ds(lo, sz)], sem))
        for r in range(nb_w):
            pltpu.sync_copy(idx_hbm.at[pl.ds(base + r * _R, _R)], idx_v2.at[r])
        pltpu.sync_copy(idx_v2, fidx_hbm.at[pl.ds(wid * nb_w, nb_w)])
        for c in copies:
            c.wait()
        pltpu.sync_copy(rows_v, out_hbm.at[pl.ds(base, b_per_w)])

    return k(emb, idx)


def _rankorth_body(w_ref, orth_ref, rank_ref):
    w = w_ref[...]
    g = lax.dot_general(w, w, (((1,), (1,)), ((), ())),
                        precision=lax.Precision.HIGHEST,
                        preferred_element_type=jnp.float32)
    subl = lax.broadcasted_iota(jnp.int32, (_D, _D), 0)
    lane = lax.broadcasted_iota(jnp.int32, (_D, _D), 1)
    eye = jnp.where(subl == lane, 1.0, 0.0).astype(jnp.float32)
    gi = g - eye
    orth_ref[0, 0] = jnp.sqrt(jnp.sum(gi * gi))

    def piter(_, carry):
        vrow, lam = carry
        vcol = jnp.sum(g * vrow, axis=1, keepdims=True)
        vcol = vcol / jnp.max(jnp.abs(vcol))
        vrow2 = jnp.sum(g * vcol, axis=0, keepdims=True)
        lam2 = jnp.max(jnp.abs(vrow2))
        return (vrow2 / lam2, lam2)
    _, lam = lax.fori_loop(
        0, 8, piter, (jnp.ones((1, _D), jnp.float32),
                      jnp.asarray(1.0, jnp.float32)))

    tau = lam * _TOLC
    a0 = g - tau * eye
    lane1 = lax.broadcasted_iota(jnp.int32, (1, _D), 1)
    sub1 = lax.broadcasted_iota(jnp.int32, (_D, 1), 0)

    def ldl_step(j, carry):
        a, cnt = carry
        lanej = lane == j
        sublj = subl == j
        d = jnp.sum(jnp.where(lanej & sublj, a, 0.0))
        row = jnp.sum(jnp.where(sublj, a, 0.0), axis=0, keepdims=True)
        col = jnp.sum(jnp.where(lanej, a, 0.0), axis=1, keepdims=True)
        rowm = jnp.where(lane1 > j, row, 0.0)
        colm = jnp.where(sub1 > j, col, 0.0)
        dsafe = jnp.where(d == 0.0, jnp.asarray(-1e-30, jnp.float32), d)
        a = a - colm * (rowm / dsafe)
        cnt = cnt + (d > 0.0).astype(jnp.int32)
        return (a, cnt)
    _, cnt = lax.fori_loop(0, _D, ldl_step, (a0, jnp.asarray(0, jnp.int32)))
    rank_ref[0, 0] = cnt.astype(jnp.float32)


def _rankorth_call(weight):
    scal = jax.ShapeDtypeStruct((1, 1), jnp.float32)
    smem_out = pl.BlockSpec(memory_space=pltpu.SMEM)
    return pl.pallas_call(
        _rankorth_body,
        out_specs=[smem_out, smem_out],
        out_shape=[scal, scal],
    )(weight)


def _loss_step(x_ref, q_ref, d4_sqrt_acc):
    diff = q_ref[...] - x_ref[...]
    d2 = diff * diff
    d4 = d2 * d2
    c = jnp.float32(0.0)
    for bi in range(_NB):
        s = jnp.sum(d4[bi * _R:(bi + 1) * _R], axis=0)
        c = c + jnp.sum(jnp.sqrt(s))
    d4_sqrt_acc[0] += c


def _loss_part_body(x_ref, q_ref, accin_ref, accout_ref, acc_ref):
    step = pl.program_id(0)

    @pl.when(step == 0)
    def _init():
        acc_ref[0] = accin_ref[0, 0]

    _loss_step(x_ref, q_ref, acc_ref)

    @pl.when(step == pl.num_programs(0) - 1)
    def _fin():
        accout_ref[0, 0] = acc_ref[0]


def _loss_final_body(x_ref, q_ref, accin_ref, orth_ref, vq_ref, ql_ref, acc_ref):
    step = pl.program_id(0)

    @pl.when(step == 0)
    def _init():
        acc_ref[0] = accin_ref[0, 0]

    _loss_step(x_ref, q_ref, acc_ref)

    @pl.when(step == pl.num_programs(0) - 1)
    def _fin():
        vq = acc_ref[0] / (_N * _D)
        vq_ref[0, 0] = vq
        ql_ref[0, 0] = _LAM_VQ * vq + _LAM_COMMIT * vq + _LAM_ORTH * orth_ref[0, 0]


def _loss_call(flat, quant_part, row0, acc_in, orth=None):
    rows = quant_part.shape[0]
    grid = rows // (_NB * _R)
    blk0 = row0 // (_NB * _R)
    scal = jax.ShapeDtypeStruct((1, 1), jnp.float32)
    smem = pl.BlockSpec(memory_space=pltpu.SMEM)
    in_specs = [
        pl.BlockSpec((_NB * _R, _D), lambda i: (i + blk0, 0)),
        pl.BlockSpec((_NB * _R, _D), lambda i: (i, 0)),
        smem,
    ]
    if orth is None:
        return pl.pallas_call(
            _loss_part_body,
            grid=(grid,),
            in_specs=in_specs,
            out_specs=smem,
            out_shape=scal,
            scratch_shapes=[pltpu.SMEM((1,), jnp.float32)],
        )(flat, quant_part, acc_in)
    return pl.pallas_call(
        _loss_final_body,
        grid=(grid,),
        in_specs=in_specs + [smem],
        out_specs=[smem, smem],
        out_shape=[scal, scal],
        scratch_shapes=[pltpu.SMEM((1,), jnp.float32)],
    )(flat, quant_part, acc_in, orth)


def kernel(soft_fillers, weight):
    n, r, d = soft_fillers.shape
    flat = soft_fillers.reshape(n * r, d)
    emb = weight.T
    idx = _argmin_call(flat, weight)
    quant, fidx = _sc_gather(emb, idx)
    orth, rank = _rankorth_call(weight)
    zero = jnp.zeros((1, 1), jnp.float32)
    vq, ql = _loss_call(flat, quant, 0, zero, orth)
    quant3 = quant.reshape(n, r, d)
    return (ql[0, 0], vq[0, 0], vq[0, 0], orth[0, 0], rank[0, 0],
            quant3, quant3, fidx)


# speedup vs baseline: 1.0788x; 1.0396x over previous
import functools

import jax
import jax.numpy as jnp
from jax import lax
from jax.experimental import pallas as pl
from jax.experimental.pallas import tpu as pltpu
from jax.experimental.pallas import tpu_sc as plsc

_N = 64
_R = 576
_D = 64
_K = 1024
_NTOK = _N * _R

_BM = 6144
_GRID = _NTOK // _BM

_NB = 8
_LGRID = _N // _NB

_LAM_VQ = 1.0
_LAM_COMMIT = 0.5
_LAM_ORTH = 0.0

_EPS32 = 1.1920929e-07
_TOLC = (_K * _EPS32) ** 2


def _argmin_body(x_ref, w_ref, o_ref):
    x = x_ref[...]
    w = w_ref[...]
    m2 = lax.dot_general(x, w * -2.0, (((1,), (0,)), ((), ())),
                         preferred_element_type=jnp.float32)
    x2 = jnp.sum(x * x, axis=1, keepdims=True)
    e2 = jnp.sum(w * w, axis=0, keepdims=True)
    dist = x2 + m2 + e2
    mn = jnp.min(dist, axis=1, keepdims=True)
    lanef = lax.broadcasted_iota(jnp.int32, dist.shape, 1).astype(jnp.float32)
    idxf = jnp.min(jnp.where(dist == mn, lanef, jnp.float32(_K)), axis=1)
    o_ref[...] = idxf.astype(jnp.int32).reshape(_BM // 128, 128)


def _argmin_call(flat, weight):
    ntok = flat.shape[0]
    idx2 = pl.pallas_call(
        _argmin_body,
        grid=(ntok // _BM,),
        in_specs=[
            pl.BlockSpec((_BM, _D), lambda i: (i, 0)),
            pl.BlockSpec((_D, _K), lambda i: (0, 0)),
        ],
        out_specs=pl.BlockSpec((_BM // 128, 128), lambda i: (i, 0)),
        out_shape=jax.ShapeDtypeStruct((ntok // 128, 128), jnp.int32),
    )(flat, weight)
    return idx2.reshape(-1)


def _sc_gather(emb, idx):
    info = plsc.get_sparse_core_info()
    nc, ns = info.num_cores, info.num_subcores
    nw = nc * ns
    b = idx.shape[0]
    b_per_w = b // nw
    ch = 128
    n_ch = -(-b_per_w // ch)
    mesh = plsc.VectorSubcoreMesh(core_axis_name="c", subcore_axis_name="s")

    nb_w = b_per_w // _R

    @functools.partial(
        pl.kernel, mesh=mesh,
        compiler_params=pltpu.CompilerParams(use_tc_tiling_on_sc=False),
        out_type=[jax.ShapeDtypeStruct((b, _D), jnp.float32),
                  jax.ShapeDtypeStruct((b // _R, _R), jnp.int32)],
        scratch_types=[
            pltpu.VMEM((b_per_w,), jnp.int32),
            pltpu.VMEM((nb_w, _R), jnp.int32),
            pltpu.VMEM((b_per_w, _D), jnp.float32),
            pltpu.SemaphoreType.DMA,
        ],
    )
    def k(emb_hbm, idx_hbm, out_hbm, fidx_hbm, idx_v, idx_v2, rows_v, sem):
        wid = lax.axis_index("s") * nc + lax.axis_index("c")
        base = wid * b_per_w
        pltpu.sync_copy(idx_hbm.at[pl.ds(base, b_per_w)], idx_v)
        copies = []
        for j in range(n_ch):
            lo = j * ch
            sz = min(ch, b_per_w - lo)
            copies.append(pltpu.async_copy(
                emb_hbm.at[idx_v.at[pl.ds(lo, sz)]],
                rows_v.at[pl.ds(lo, sz)], sem))
        for r in range(nb_w):
            pltpu.sync_copy(idx_hbm.at[pl.ds(base + r * _R, _R)], idx_v2.at[r])
        pltpu.sync_copy(idx_v2, fidx_hbm.at[pl.ds(wid * nb_w, nb_w)])
        for c in copies:
            c.wait()
        pltpu.sync_copy(rows_v, out_hbm.at[pl.ds(base, b_per_w)])

    return k(emb, idx)


def _rankorth_body(w_ref, orth_ref, rank_ref):
    w = w_ref[...]
    g = lax.dot_general(w, w, (((1,), (1,)), ((), ())),
                        precision=lax.Precision.HIGHEST,
                        preferred_element_type=jnp.float32)
    subl = lax.broadcasted_iota(jnp.int32, (_D, _D), 0)
    lane = lax.broadcasted_iota(jnp.int32, (_D, _D), 1)
    eye = jnp.where(subl == lane, 1.0, 0.0).astype(jnp.float32)
    gi = g - eye
    orth_ref[0, 0] = jnp.sqrt(jnp.sum(gi * gi))

    def piter(_, carry):
        vrow, lam = carry
        vcol = jnp.sum(g * vrow, axis=1, keepdims=True)
        vcol = vcol / jnp.max(jnp.abs(vcol))
        vrow2 = jnp.sum(g * vcol, axis=0, keepdims=True)
        lam2 = jnp.max(jnp.abs(vrow2))
        return (vrow2 / lam2, lam2)
    _, lam = lax.fori_loop(
        0, 8, piter, (jnp.ones((1, _D), jnp.float32),
                      jnp.asarray(1.0, jnp.float32)))

    tau = lam * _TOLC
    a0 = g - tau * eye
    lane1 = lax.broadcasted_iota(jnp.int32, (1, _D), 1)
    sub1 = lax.broadcasted_iota(jnp.int32, (_D, 1), 0)

    def ldl_step(j, carry):
        a, cnt = carry
        lanej = lane == j
        sublj = subl == j
        d = jnp.sum(jnp.where(lanej & sublj, a, 0.0))
        row = jnp.sum(jnp.where(sublj, a, 0.0), axis=0, keepdims=True)
        col = jnp.sum(jnp.where(lanej, a, 0.0), axis=1, keepdims=True)
        rowm = jnp.where(lane1 > j, row, 0.0)
        colm = jnp.where(sub1 > j, col, 0.0)
        dsafe = jnp.where(d == 0.0, jnp.asarray(-1e-30, jnp.float32), d)
        a = a - colm * (rowm / dsafe)
        cnt = cnt + (d > 0.0).astype(jnp.int32)
        return (a, cnt)
    _, cnt = lax.fori_loop(0, _D, ldl_step, (a0, jnp.asarray(0, jnp.int32)))
    rank_ref[0, 0] = cnt.astype(jnp.float32)


def _rankorth_call(weight):
    scal = jax.ShapeDtypeStruct((1, 1), jnp.float32)
    smem_out = pl.BlockSpec(memory_space=pltpu.SMEM)
    return pl.pallas_call(
        _rankorth_body,
        out_specs=[smem_out, smem_out],
        out_shape=[scal, scal],
    )(weight)


def _loss_step(x_ref, q_ref, d4_sqrt_acc):
    diff = q_ref[...] - x_ref[...]
    d2 = diff * diff
    d4 = d2 * d2
    c = jnp.float32(0.0)
    for bi in range(_NB):
        s = jnp.sum(d4[bi * _R:(bi + 1) * _R], axis=0)
        c = c + jnp.sum(jnp.sqrt(s))
    d4_sqrt_acc[0] += c


def _loss_part_body(x_ref, q_ref, accin_ref, accout_ref, acc_ref):
    step = pl.program_id(0)

    @pl.when(step == 0)
    def _init():
        acc_ref[0] = accin_ref[0, 0]

    _loss_step(x_ref, q_ref, acc_ref)

    @pl.when(step == pl.num_programs(0) - 1)
    def _fin():
        accout_ref[0, 0] = acc_ref[0]


def _loss_final_body(x_ref, q_ref, accin_ref, orth_ref, vq_ref, ql_ref, acc_ref):
    step = pl.program_id(0)

    @pl.when(step == 0)
    def _init():
        acc_ref[0] = accin_ref[0, 0]

    _loss_step(x_ref, q_ref, acc_ref)

    @pl.when(step == pl.num_programs(0) - 1)
    def _fin():
        vq = acc_ref[0] / (_N * _D)
        vq_ref[0, 0] = vq
        ql_ref[0, 0] = _LAM_VQ * vq + _LAM_COMMIT * vq + _LAM_ORTH * orth_ref[0, 0]


def _loss_call(flat, quant_part, row0, acc_in, orth=None):
    rows = quant_part.shape[0]
    grid = rows // (_NB * _R)
    blk0 = row0 // (_NB * _R)
    scal = jax.ShapeDtypeStruct((1, 1), jnp.float32)
    smem = pl.BlockSpec(memory_space=pltpu.SMEM)
    in_specs = [
        pl.BlockSpec((_NB * _R, _D), lambda i: (i + blk0, 0)),
        pl.BlockSpec((_NB * _R, _D), lambda i: (i, 0)),
        smem,
    ]
    if orth is None:
        return pl.pallas_call(
            _loss_part_body,
            grid=(grid,),
            in_specs=in_specs,
            out_specs=smem,
            out_shape=scal,
            scratch_shapes=[pltpu.SMEM((1,), jnp.float32)],
        )(flat, quant_part, acc_in)
    return pl.pallas_call(
        _loss_final_body,
        grid=(grid,),
        in_specs=in_specs + [smem],
        out_specs=[smem, smem],
        out_shape=[scal, scal],
        scratch_shapes=[pltpu.SMEM((1,), jnp.float32)],
    )(flat, quant_part, acc_in, orth)


def kernel(soft_fillers, weight):
    n, r, d = soft_fillers.shape
    flat = soft_fillers.reshape(n * r, d)
    emb = weight.T
    idx = _argmin_call(flat, weight)
    quant, fidx = _sc_gather(emb, idx)
    orth, rank = _rankorth_call(weight)
    zero = jnp.zeros((1, 1), jnp.float32)
    vq, ql = _loss_call(flat, quant, 0, zero, orth)
    quant3 = quant.reshape(n, r, d)
    return (ql[0, 0], vq[0, 0], vq[0, 0], orth[0, 0], rank[0, 0],
            quant3, quant3, fidx)
